# Initial kernel scaffold; baseline (speedup 1.0000x reference)
import jax
import jax.numpy as jnp
from jax.experimental import pallas as pl


def kernel(x, table, pos_table):
    raise NotImplementedError("write your pallas kernel here")


# speedup vs baseline: 1.4270x; 1.4270x over previous
import functools

import jax
import jax.numpy as jnp
from jax import lax
from jax.experimental import pallas as pl
from jax.experimental.pallas import tpu as pltpu
from jax.experimental.pallas import tpu_sc as plsc

B = 4096
L = 200
EMBED = 32
N = B * L

NUM_CORES = 2
NUM_SUBCORES = 16
NW = NUM_CORES * NUM_SUBCORES

PER_W = N // NW
CHUNK_PERIODS = 8
CHUNK = CHUNK_PERIODS * L
NCHUNK = PER_W // CHUNK


def _body(x_hbm, table_hbm, pos_hbm, out_hbm, idx_v, rows_v, pos_v, sem):
    wid = lax.axis_index("s") * NUM_CORES + lax.axis_index("c")
    wbase = wid * PER_W

    pltpu.sync_copy(pos_hbm, pos_v)

    def chunk_body(c, _):
        base = wbase + c * CHUNK
        pltpu.sync_copy(x_hbm.at[pl.ds(base, CHUNK)], idx_v)
        pltpu.async_copy(table_hbm.at[idx_v], rows_v, sem).wait()

        def add_body(j, _):
            p_lo = pos_v[j, pl.ds(0, 16)]
            p_hi = pos_v[j, pl.ds(16, 16)]
            for p in range(CHUNK_PERIODS):
                r = p * L + j
                rows_v[r, pl.ds(0, 16)] += p_lo
                rows_v[r, pl.ds(16, 16)] += p_hi
            return 0

        lax.fori_loop(0, L, add_body, 0)
        pltpu.sync_copy(rows_v, out_hbm.at[pl.ds(base, CHUNK)])
        return 0

    lax.fori_loop(0, NCHUNK, chunk_body, 0)


@jax.jit
def _embed(x_flat, table, pos_table):
    mesh = plsc.VectorSubcoreMesh(
        core_axis_name="c", subcore_axis_name="s",
        num_cores=NUM_CORES, num_subcores=NUM_SUBCORES,
    )
    run = functools.partial(
        pl.kernel,
        out_type=jax.ShapeDtypeStruct((N, EMBED), jnp.float32),
        mesh=mesh,
        scratch_types=[
            pltpu.VMEM((CHUNK,), jnp.int32),
            pltpu.VMEM((CHUNK, EMBED), jnp.float32),
            pltpu.VMEM((L, EMBED), jnp.float32),
            pltpu.SemaphoreType.DMA,
        ],
        compiler_params=pltpu.CompilerParams(use_tc_tiling_on_sc=False),
    )(_body)
    return run(x_flat, table, pos_table)


def kernel(x, table, pos_table):
    x_flat = x.reshape(-1).astype(jnp.int32)
    out = _embed(x_flat, table, pos_table)
    return out.reshape(B, L, EMBED)


# speedup vs baseline: 1.4536x; 1.0187x over previous
import functools

import jax
import jax.numpy as jnp
from jax import lax
from jax.experimental import pallas as pl
from jax.experimental.pallas import tpu as pltpu
from jax.experimental.pallas import tpu_sc as plsc

B = 4096
L = 200
EMBED = 32

NUM_CORES = 2
NUM_SUBCORES = 16
NW = NUM_CORES * NUM_SUBCORES
BW = B // NW


def _body(x_hbm, table_hbm, pos_hbm, out_hbm,
          idx_all, r0, r1, t0, t1, pos_v,
          gsem0, gsem1, osem0, osem1):
    wid = lax.axis_index("s") * NUM_CORES + lax.axis_index("c")

    pltpu.sync_copy(pos_hbm, pos_v)
    pltpu.sync_copy(x_hbm.at[:, pl.ds(wid * BW, BW)], idx_all)

    iota = lax.iota(jnp.int32, 16)
    eh_lo = iota >> 3
    eh_hi = eh_lo + 2
    el = iota & 7

    rbufs = (r0, r1)
    tbufs = (t0, t1)
    gsems = (gsem0, gsem1)
    osems = (osem0, osem1)

    pltpu.async_copy(table_hbm.at[idx_all.at[0]], r0, gsem0)
    pltpu.async_copy(table_hbm.at[idx_all.at[1]], r1, gsem1)

    def step(i, _):
        for par in (0, 1):
            l = 2 * i + par
            rv, tv = rbufs[par], tbufs[par]
            gsem, osem = gsems[par], osems[par]

            pltpu.make_async_copy(table_hbm.at[idx_all.at[l]], rv, gsem).wait()
            @pl.when(i >= 1)
            def _():
                pltpu.make_async_copy(tv, out_hbm.at[l, :, wid], osem).wait()

            p_lo = pos_v[l, pl.ds(0, 16)]
            p_hi = pos_v[l, pl.ds(16, 16)]

            def tok(b, _):
                bl = jnp.full((16,), b, jnp.int32)
                v_lo = rv[b, pl.ds(0, 16)] + p_lo
                v_hi = rv[b, pl.ds(16, 16)] + p_hi
                plsc.store_scatter(tv, [eh_lo, el, bl], v_lo)
                plsc.store_scatter(tv, [eh_hi, el, bl], v_hi)
                return 0

            lax.fori_loop(0, BW, tok, 0)

            pltpu.async_copy(tv, out_hbm.at[l, :, wid], osem)

            @pl.when(i < (L // 2) - 1)
            def _():
                pltpu.async_copy(table_hbm.at[idx_all.at[l + 2]], rv, gsem)
        return 0

    lax.fori_loop(0, L // 2, step, 0)

    pltpu.make_async_copy(t0, out_hbm.at[L - 2, :, wid], osem0).wait()
    pltpu.make_async_copy(t1, out_hbm.at[L - 1, :, wid], osem1).wait()


@jax.jit
def _embed(x_t, table, pos_table):
    mesh = plsc.VectorSubcoreMesh(
        core_axis_name="c", subcore_axis_name="s",
        num_cores=NUM_CORES, num_subcores=NUM_SUBCORES,
    )
    run = functools.partial(
        pl.kernel,
        out_type=jax.ShapeDtypeStruct((L, EMBED // 8, NW, 8, BW), jnp.float32),
        mesh=mesh,
        scratch_types=[
            pltpu.VMEM((L, BW), jnp.int32),
            pltpu.VMEM((BW, EMBED), jnp.float32),
            pltpu.VMEM((BW, EMBED), jnp.float32),
            pltpu.VMEM((EMBED // 8, 8, BW), jnp.float32),
            pltpu.VMEM((EMBED // 8, 8, BW), jnp.float32),
            pltpu.VMEM((L, EMBED), jnp.float32),
            pltpu.SemaphoreType.DMA,
            pltpu.SemaphoreType.DMA,
            pltpu.SemaphoreType.DMA,
            pltpu.SemaphoreType.DMA,
        ],
        compiler_params=pltpu.CompilerParams(
            use_tc_tiling_on_sc=False, needs_layout_passes=False),
    )(_body)
    return run(x_t, table, pos_table)


def kernel(x, table, pos_table):
    arr = _embed(x.T.astype(jnp.int32), table, pos_table)
    return arr.transpose(2, 4, 0, 1, 3).reshape(B, L, EMBED)
